# Initial kernel scaffold; baseline (speedup 1.0000x reference)
import jax
import jax.numpy as jnp
from jax import lax
from jax.experimental import pallas as pl
from jax.experimental.pallas import tpu as pltpu
from jax.experimental.pallas import tpu_sc as plsc

N = 10000
E = 320000
F = 128
ALPHA = 0.2

NC = 2
NS = 16
NT = NC * NS
EC = E // NT
B = 80
NB = EC // B
NBUF = 3
ROWS_PER_TILE = N // NS

ROW_BLK = 1000
GRID = N // ROW_BLK


def _mm_body(x_ref, w_ref, a_ref, wh_ref, s1_ref, s2_ref):
    xb = x_ref[...]
    wh = jnp.dot(xb, w_ref[...], preferred_element_type=jnp.float32)
    wh_ref[...] = wh
    s1_ref[...] = jnp.dot(wh, a_ref[:F], preferred_element_type=jnp.float32)
    s2_ref[...] = jnp.dot(wh, a_ref[F:], preferred_element_type=jnp.float32)


def _stage1(x, W, a):
    return pl.pallas_call(
        _mm_body,
        grid=(GRID,),
        in_specs=[
            pl.BlockSpec((ROW_BLK, F), lambda i: (i, 0)),
            pl.BlockSpec((F, F), lambda i: (0, 0)),
            pl.BlockSpec((2 * F, 1), lambda i: (0, 0)),
        ],
        out_specs=[
            pl.BlockSpec((ROW_BLK, F), lambda i: (i, 0)),
            pl.BlockSpec((ROW_BLK, 1), lambda i: (i, 0)),
            pl.BlockSpec((ROW_BLK, 1), lambda i: (i, 0)),
        ],
        out_shape=[
            jax.ShapeDtypeStruct((N, F), jnp.float32),
            jax.ShapeDtypeStruct((N, 1), jnp.float32),
            jax.ShapeDtypeStruct((N, 1), jnp.float32),
        ],
    )(x, W, a)


def _sc_body(wh_hbm, s1_hbm, s2_hbm, src_hbm, dst_hbm, dstw_hbm,
             acc_out, den_out,
             s1_v, s2_v, src_v, dst_v, dstw_v, ex_v, den_v, rows_v, sems,
             acc_s):
    c = lax.axis_index("c")
    s = lax.axis_index("s")
    g = c * NS + s
    base = g * EC

    pltpu.sync_copy(s1_hbm, s1_v)
    pltpu.sync_copy(s2_hbm, s2_v)
    pltpu.sync_copy(src_hbm.at[pl.ds(base, EC)], src_v)
    pltpu.sync_copy(dst_hbm.at[pl.ds(base, EC)], dst_v)
    pltpu.sync_copy(dstw_hbm.at[g], dstw_v)

    zv = jnp.zeros((16,), jnp.float32)

    def _zrow(r, _):
        for f in range(F // 16):
            rows_v[0, r, pl.ds(f * 16, 16)] = zv
        return 0
    lax.fori_loop(0, B, _zrow, 0)
    row0 = s * ROWS_PER_TILE
    nfull = ROWS_PER_TILE // B
    rem = ROWS_PER_TILE - nfull * B
    for t in range(nfull):
        pltpu.sync_copy(rows_v.at[0], acc_s.at[pl.ds(row0 + t * B, B)])
    pltpu.sync_copy(rows_v.at[0, pl.ds(0, rem)],
                    acc_s.at[pl.ds(row0 + nfull * B, rem)])

    def _zden(i, _):
        den_v[pl.ds(i * 16, 16)] = zv
        return 0
    lax.fori_loop(0, N // 16, _zden, 0)

    def _score(i, _):
        d16 = dst_v[pl.ds(i * 16, 16)]
        t16 = src_v[pl.ds(i * 16, 16)]
        e = plsc.load_gather(s1_v, [d16]) + plsc.load_gather(s2_v, [t16])
        e = jnp.where(e >= 0.0, e, ALPHA * e)
        ex = jnp.exp(e)
        ex_v[pl.ds(i * 16, 16)] = ex
        plsc.addupdate_scatter(den_v, [d16], ex)
        return 0
    lax.fori_loop(0, EC // 16, _score, 0)
    pltpu.sync_copy(den_v, den_out.at[g])

    plsc.subcore_barrier()

    def _issue(b, k):
        idx = src_v.at[pl.ds(b * B, B)]
        pltpu.async_copy(wh_hbm.at[idx], rows_v.at[k], sems.at[k])

    for k in range(NBUF):
        _issue(k, k)

    def _scale_rows(k, b):
        def _edge(kk, _):
            eidx = b * B + kk
            exb = plsc.load_gather(ex_v, [jnp.full((16,), eidx, jnp.int32)])
            for f in range(F // 16):
                sl = pl.ds(f * 16, 16)
                rows_v[k, kk, sl] = rows_v[k, kk, sl] * exb
            return 0
        lax.fori_loop(0, B, _edge, 0)

    def _outer(bb, _):
        for k in range(NBUF):
            b = bb * NBUF + k

            @pl.when(b < NB)
            def _():
                pltpu.make_async_copy(wh_hbm.at[src_v.at[pl.ds(b * B, B)]],
                                      rows_v.at[k], sems.at[k]).wait()
                _scale_rows(k, b)
                pltpu.sync_copy(rows_v.at[k], acc_s.at[dstw_v.at[b]],
                                add=True)

                @pl.when(b + NBUF < NB)
                def _():
                    _issue(b + NBUF, k)
        return 0
    lax.fori_loop(0, (NB + NBUF - 1) // NBUF, _outer, 0)

    plsc.subcore_barrier()
    pltpu.sync_copy(acc_s.at[pl.ds(row0, ROWS_PER_TILE)],
                    acc_out.at[c, pl.ds(row0, ROWS_PER_TILE)])


def _stage2(wh, s1, s2, src, dst, dstw):
    mesh = plsc.VectorSubcoreMesh(core_axis_name="c", subcore_axis_name="s",
                                  num_cores=NC, num_subcores=NS)
    return pl.kernel(
        _sc_body,
        out_type=[
            jax.ShapeDtypeStruct((NC, N, F), jnp.float32),
            jax.ShapeDtypeStruct((NT, N), jnp.float32),
        ],
        mesh=mesh,
        scratch_types=[
            pltpu.VMEM((N,), jnp.float32),
            pltpu.VMEM((N,), jnp.float32),
            pltpu.VMEM((EC,), jnp.int32),
            pltpu.VMEM((EC,), jnp.int32),
            pltpu.VMEM((NB, B), jnp.int32),
            pltpu.VMEM((EC,), jnp.float32),
            pltpu.VMEM((N,), jnp.float32),
            pltpu.VMEM((NBUF, B, F), jnp.float32),
            pltpu.SemaphoreType.DMA((NBUF,)),
            pltpu.VMEM_SHARED((N, F), jnp.float32),
        ],
    )(wh, s1, s2, src, dst, dstw)


def _fin_body(acc_ref, den_ref, out_ref):
    acc = acc_ref[0] + acc_ref[1]
    den = jnp.sum(den_ref[...], axis=0)
    o = acc / (den[:, None] + 1e-16)
    out_ref[...] = jnp.where(o > 0.0, o, jnp.expm1(o))


def _stage3(acc, den):
    return pl.pallas_call(
        _fin_body,
        grid=(GRID,),
        in_specs=[
            pl.BlockSpec((NC, ROW_BLK, F), lambda i: (0, i, 0)),
            pl.BlockSpec((NT, ROW_BLK), lambda i: (0, i)),
        ],
        out_specs=pl.BlockSpec((ROW_BLK, F), lambda i: (i, 0)),
        out_shape=jax.ShapeDtypeStruct((N, F), jnp.float32),
    )(acc, den)


def kernel(x, edge_index, W, a):
    src = edge_index[0]
    dst = edge_index[1]
    wh, s1, s2 = _stage1(x, W, a)
    dstw = dst.reshape(NT, NB, B)
    acc, den_parts = _stage2(wh, s1.reshape(N), s2.reshape(N), src, dst, dstw)
    return _stage3(acc, den_parts)


# speedup vs baseline: 38.1749x; 38.1749x over previous
import jax
import jax.numpy as jnp
from jax import lax
from jax.experimental import pallas as pl
from jax.experimental.pallas import tpu as pltpu
from jax.experimental.pallas import tpu_sc as plsc

N = 10000
E = 320000
F = 128
ALPHA = 0.2

NC = 2
NS = 16
NT = NC * NS
EC = E // NT
B = 80
NBUF = 3
NP = 10240
ROWS_PER_TILE = NP // NS

MM_BLK = 1000
MM_GRID = N // MM_BLK
ROW_BLK = 1024
GRID = NP // ROW_BLK


def _mm_body(x_ref, w_ref, a_ref, wh_ref, s1_ref, s2_ref):
    xb = x_ref[...]
    wh = jnp.dot(xb, w_ref[...], preferred_element_type=jnp.float32)
    wh_ref[...] = wh
    s1_ref[...] = jnp.dot(wh, a_ref[:F], preferred_element_type=jnp.float32)
    s2_ref[...] = jnp.dot(wh, a_ref[F:], preferred_element_type=jnp.float32)


def _stage1(x, W, a):
    return pl.pallas_call(
        _mm_body,
        grid=(MM_GRID,),
        in_specs=[
            pl.BlockSpec((MM_BLK, F), lambda i: (i, 0)),
            pl.BlockSpec((F, F), lambda i: (0, 0)),
            pl.BlockSpec((2 * F, 1), lambda i: (0, 0)),
        ],
        out_specs=[
            pl.BlockSpec((MM_BLK, F), lambda i: (i, 0)),
            pl.BlockSpec((MM_BLK, 1), lambda i: (i, 0)),
            pl.BlockSpec((MM_BLK, 1), lambda i: (i, 0)),
        ],
        out_shape=[
            jax.ShapeDtypeStruct((N, F), jnp.float32),
            jax.ShapeDtypeStruct((N, 1), jnp.float32),
            jax.ShapeDtypeStruct((N, 1), jnp.float32),
        ],
    )(x, W, a)


def _sc_score_body(s1_hbm, s2_hbm, src_hbm, dst_hbm,
                   ex_out, den_out,
                   s1_v, s2_v, src_v, dst_v, ex_v, den_v):
    c = lax.axis_index("c")
    s = lax.axis_index("s")
    g = c * NS + s
    base = g * EC

    pltpu.sync_copy(s1_hbm, s1_v)
    pltpu.sync_copy(s2_hbm, s2_v)
    pltpu.sync_copy(src_hbm.at[pl.ds(base, EC)], src_v)
    pltpu.sync_copy(dst_hbm.at[pl.ds(base, EC)], dst_v)

    zv = jnp.zeros((16,), jnp.float32)

    def _zden(i, _):
        den_v[pl.ds(i * 16, 16)] = zv
        return 0
    lax.fori_loop(0, NP // 16, _zden, 0)

    def _score(i, _):
        d16 = dst_v[pl.ds(i * 16, 16)]
        t16 = src_v[pl.ds(i * 16, 16)]
        e = plsc.load_gather(s1_v, [d16]) + plsc.load_gather(s2_v, [t16])
        e = jnp.where(e >= 0.0, e, ALPHA * e)
        ex = jnp.exp(e)
        ex_v[pl.ds(i * 16, 16)] = ex
        plsc.addupdate_scatter(den_v, [d16], ex)
        return 0
    lax.fori_loop(0, EC // 16, _score, 0)

    pltpu.sync_copy(ex_v, ex_out.at[pl.ds(base, EC)])
    pltpu.sync_copy(den_v, den_out.at[pl.ds(g * NP, NP)])


def _stage2a(s1, s2, src, dst):
    mesh = plsc.VectorSubcoreMesh(core_axis_name="c", subcore_axis_name="s",
                                  num_cores=NC, num_subcores=NS)
    return pl.kernel(
        _sc_score_body,
        out_type=[
            jax.ShapeDtypeStruct((E,), jnp.float32),
            jax.ShapeDtypeStruct((NT * NP,), jnp.float32),
        ],
        mesh=mesh,
        compiler_params=pltpu.CompilerParams(needs_layout_passes=False),
        scratch_types=[
            pltpu.VMEM((N,), jnp.float32),
            pltpu.VMEM((N,), jnp.float32),
            pltpu.VMEM((EC,), jnp.int32),
            pltpu.VMEM((EC,), jnp.int32),
            pltpu.VMEM((EC,), jnp.float32),
            pltpu.VMEM((NP,), jnp.float32),
        ],
    )(s1, s2, src, dst)


CH = 2000
NCH = EC // CH
NBC = CH // B


def _sc_rows_body(wh_hbm, src_hbm, dstw_hbm, ex_hbm,
                  acc_out,
                  src_c, dstw_c, ex_c, rows_v, sems,
                  acc_s):
    c = lax.axis_index("c")
    s = lax.axis_index("s")
    g = c * NS + s
    base = g * EC

    zv = jnp.zeros((16,), jnp.float32)

    def _zrow(r, _):
        for f in range(F // 16):
            rows_v[0, r, pl.ds(f * 16, 16)] = zv
        return 0
    lax.fori_loop(0, B, _zrow, 0)
    row0 = s * ROWS_PER_TILE
    for t in range(ROWS_PER_TILE // B):
        pltpu.sync_copy(rows_v.at[0], acc_s.at[pl.ds(row0 + t * B, B)])

    plsc.subcore_barrier()

    def _issue(b, k):
        idx = src_c.at[pl.ds(b * B, B)]
        pltpu.async_copy(wh_hbm.at[idx], rows_v.at[k], sems.at[k])

    def _scale_rows(k, b):
        def _edge(kk, _):
            eidx = b * B + kk
            exb = plsc.load_gather(ex_c, [jnp.full((16,), eidx, jnp.int32)])
            for f in range(F // 16):
                sl = pl.ds(f * 16, 16)
                rows_v[k, kk, sl] = rows_v[k, kk, sl] * exb
            return 0
        lax.fori_loop(0, B, _edge, 0)

    def _chunk(ch, _):
        pltpu.sync_copy(src_hbm.at[pl.ds(base + ch * CH, CH)], src_c)
        pltpu.sync_copy(ex_hbm.at[pl.ds(base + ch * CH, CH)], ex_c)
        pltpu.sync_copy(dstw_hbm.at[g, ch], dstw_c)

        for k in range(NBUF):
            _issue(k, k)

        def _outer(bb, _):
            for k in range(NBUF):
                b = bb * NBUF + k

                @pl.when(b < NBC)
                def _():
                    pltpu.make_async_copy(
                        wh_hbm.at[src_c.at[pl.ds(b * B, B)]],
                        rows_v.at[k], sems.at[k]).wait()
                    _scale_rows(k, b)
                    pltpu.sync_copy(rows_v.at[k], acc_s.at[dstw_c.at[b]],
                                    add=True)

                    @pl.when(b + NBUF < NBC)
                    def _():
                        _issue(b + NBUF, k)
            return 0
        lax.fori_loop(0, (NBC + NBUF - 1) // NBUF, _outer, 0)
        return 0
    lax.fori_loop(0, NCH, _chunk, 0)

    plsc.subcore_barrier()
    pltpu.sync_copy(acc_s.at[pl.ds(row0, ROWS_PER_TILE)],
                    acc_out.at[c, pl.ds(row0, ROWS_PER_TILE)])


def _stage2b(wh, src, dstw, ex):
    mesh = plsc.VectorSubcoreMesh(core_axis_name="c", subcore_axis_name="s",
                                  num_cores=NC, num_subcores=NS)
    return pl.kernel(
        _sc_rows_body,
        out_type=jax.ShapeDtypeStruct((NC, NP, F), jnp.float32),
        mesh=mesh,
        compiler_params=pltpu.CompilerParams(needs_layout_passes=False),
        scratch_types=[
            pltpu.VMEM((CH,), jnp.int32),
            pltpu.VMEM((NBC, B), jnp.int32),
            pltpu.VMEM((CH,), jnp.float32),
            pltpu.VMEM((NBUF, B, F), jnp.float32),
            pltpu.SemaphoreType.DMA((NBUF,)),
            pltpu.VMEM_SHARED((NP, F), jnp.float32),
        ],
    )(wh, src, dstw, ex)


def _fin_body(acc_ref, den_ref, out_ref):
    i = pl.program_id(0)
    acc = acc_ref[0] + acc_ref[1]
    den = jnp.sum(den_ref[:, pl.ds(i * ROW_BLK, ROW_BLK)], axis=0)
    o = acc / (den[:, None] + 1e-16)
    out_ref[...] = jnp.where(o > 0.0, o, jnp.exp(jnp.minimum(o, 0.0)) - 1.0)


def _stage3(acc, den):
    return pl.pallas_call(
        _fin_body,
        grid=(GRID,),
        in_specs=[
            pl.BlockSpec((NC, ROW_BLK, F), lambda i: (0, i, 0)),
            pl.BlockSpec((NT, NP), lambda i: (0, 0)),
        ],
        out_specs=pl.BlockSpec((ROW_BLK, F), lambda i: (i, 0)),
        out_shape=jax.ShapeDtypeStruct((NP, F), jnp.float32),
    )(acc, den)


def kernel(x, edge_index, W, a):
    src = edge_index[0]
    dst = edge_index[1]
    wh, s1, s2 = _stage1(x, W, a)
    ex, den_parts = _stage2a(s1.reshape(N), s2.reshape(N), src, dst)
    dstw = dst.reshape(NT, NCH, NBC, B)
    acc = _stage2b(wh, src, dstw, ex)
    return _stage3(acc, den_parts.reshape(NT, NP))[:N]


# speedup vs baseline: 42.9823x; 1.1259x over previous
import jax
import jax.numpy as jnp
from jax import lax
from jax.experimental import pallas as pl
from jax.experimental.pallas import tpu as pltpu
from jax.experimental.pallas import tpu_sc as plsc

N = 10000
E = 320000
F = 128
ALPHA = 0.2

NC = 2
NS = 16
NT = NC * NS
EC = E // NT
B = 80
NBUF = 3
NP = 10240
ROWS_PER_TILE = NP // NS

MM_BLK = 1000
MM_GRID = N // MM_BLK
ROW_BLK = 1024
GRID = NP // ROW_BLK


def _mm_body(x_ref, w_ref, a_ref, wh_ref, s1_ref, s2_ref):
    xb = x_ref[...]
    wh = jnp.dot(xb, w_ref[...], preferred_element_type=jnp.float32)
    wh_ref[...] = wh
    s1_ref[...] = jnp.dot(wh, a_ref[:F], preferred_element_type=jnp.float32)
    s2_ref[...] = jnp.dot(wh, a_ref[F:], preferred_element_type=jnp.float32)


def _stage1(x, W, a):
    return pl.pallas_call(
        _mm_body,
        grid=(MM_GRID,),
        in_specs=[
            pl.BlockSpec((MM_BLK, F), lambda i: (i, 0)),
            pl.BlockSpec((F, F), lambda i: (0, 0)),
            pl.BlockSpec((2 * F, 1), lambda i: (0, 0)),
        ],
        out_specs=[
            pl.BlockSpec((MM_BLK, F), lambda i: (i, 0)),
            pl.BlockSpec((MM_BLK, 1), lambda i: (i, 0)),
            pl.BlockSpec((MM_BLK, 1), lambda i: (i, 0)),
        ],
        out_shape=[
            jax.ShapeDtypeStruct((N, F), jnp.float32),
            jax.ShapeDtypeStruct((N, 1), jnp.float32),
            jax.ShapeDtypeStruct((N, 1), jnp.float32),
        ],
    )(x, W, a)


def _sc_score_body(s1_hbm, s2_hbm, src_hbm, dst_hbm,
                   ex_out, den_out,
                   s1_v, s2_v, src_v, dst_v, ex_v, den_v):
    c = lax.axis_index("c")
    s = lax.axis_index("s")
    g = c * NS + s
    base = g * EC

    pltpu.sync_copy(s1_hbm, s1_v)
    pltpu.sync_copy(s2_hbm, s2_v)
    pltpu.sync_copy(src_hbm.at[pl.ds(base, EC)], src_v)
    pltpu.sync_copy(dst_hbm.at[pl.ds(base, EC)], dst_v)

    zv = jnp.zeros((16,), jnp.float32)

    def _zden(i, _):
        den_v[pl.ds(i * 16, 16)] = zv
        return 0
    lax.fori_loop(0, NP // 16, _zden, 0)

    def _score(i, _):
        d16 = dst_v[pl.ds(i * 16, 16)]
        t16 = src_v[pl.ds(i * 16, 16)]
        e = plsc.load_gather(s1_v, [d16]) + plsc.load_gather(s2_v, [t16])
        e = jnp.where(e >= 0.0, e, ALPHA * e)
        ex = jnp.exp(e)
        ex_v[pl.ds(i * 16, 16)] = ex
        plsc.addupdate_scatter(den_v, [d16], ex)
        return 0
    lax.fori_loop(0, EC // 16, _score, 0)

    pltpu.sync_copy(ex_v, ex_out.at[pl.ds(base, EC)])
    pltpu.sync_copy(den_v, den_out.at[pl.ds(g * NP, NP)])


def _stage2a(s1, s2, src, dst):
    mesh = plsc.VectorSubcoreMesh(core_axis_name="c", subcore_axis_name="s",
                                  num_cores=NC, num_subcores=NS)
    return pl.kernel(
        _sc_score_body,
        out_type=[
            jax.ShapeDtypeStruct((E,), jnp.float32),
            jax.ShapeDtypeStruct((NT * NP,), jnp.float32),
        ],
        mesh=mesh,
        compiler_params=pltpu.CompilerParams(needs_layout_passes=False),
        scratch_types=[
            pltpu.VMEM((N,), jnp.float32),
            pltpu.VMEM((N,), jnp.float32),
            pltpu.VMEM((EC,), jnp.int32),
            pltpu.VMEM((EC,), jnp.int32),
            pltpu.VMEM((EC,), jnp.float32),
            pltpu.VMEM((NP,), jnp.float32),
        ],
    )(s1, s2, src, dst)


CH = 2000
NCH = EC // CH
NBC = CH // B


def _sc_rows_body(wh_hbm, src_hbm, dstw_hbm, ex_hbm,
                  acc_out,
                  src_c, dstw_c, ex_c, rows_v, sems, ssems,
                  acc_s):
    c = lax.axis_index("c")
    s = lax.axis_index("s")
    g = c * NS + s
    base = g * EC

    zv = jnp.zeros((16,), jnp.float32)

    def _zrow(r, _):
        for f in range(F // 16):
            rows_v[0, r, pl.ds(f * 16, 16)] = zv
        return 0
    lax.fori_loop(0, B, _zrow, 0)
    row0 = s * ROWS_PER_TILE
    for t in range(ROWS_PER_TILE // B):
        pltpu.sync_copy(rows_v.at[0], acc_s.at[pl.ds(row0 + t * B, B)])

    plsc.subcore_barrier()

    def _issue(b, k):
        idx = src_c.at[pl.ds(b * B, B)]
        pltpu.async_copy(wh_hbm.at[idx], rows_v.at[k], sems.at[k])

    def _scale_rows(k, b):
        def _edge(kk, _):
            eidx = b * B + kk
            exb = plsc.load_gather(ex_c, [jnp.full((16,), eidx, jnp.int32)])
            for f in range(F // 16):
                sl = pl.ds(f * 16, 16)
                rows_v[k, kk, sl] = rows_v[k, kk, sl] * exb
            return 0
        lax.fori_loop(0, B, _edge, 0)

    def _scatter(k, b):
        return pltpu.make_async_copy(rows_v.at[k], acc_s.at[dstw_c.at[b]],
                                     ssems.at[k])

    def _chunk(ch, _):
        pltpu.sync_copy(src_hbm.at[pl.ds(base + ch * CH, CH)], src_c)
        pltpu.sync_copy(ex_hbm.at[pl.ds(base + ch * CH, CH)], ex_c)
        pltpu.sync_copy(dstw_hbm.at[g, ch], dstw_c)

        for k in range(NBUF):
            _issue(k, k)

        def _outer(bb, _):
            for k in range(NBUF):
                b = bb * NBUF + k

                @pl.when(b < NBC)
                def _():
                    pltpu.make_async_copy(
                        wh_hbm.at[src_c.at[pl.ds(b * B, B)]],
                        rows_v.at[k], sems.at[k]).wait()
                    _scale_rows(k, b)
                    _scatter(k, b).start(add=True)

                    kp = (k + NBUF - 1) % NBUF

                    @pl.when(b >= 1)
                    def _():
                        _scatter(kp, b - 1).wait()

                        @pl.when(b - 1 + NBUF < NBC)
                        def _():
                            _issue(b - 1 + NBUF, kp)
            return 0
        lax.fori_loop(0, (NBC + NBUF - 1) // NBUF, _outer, 0)
        _scatter((NBC - 1) % NBUF, NBC - 1).wait()
        return 0
    lax.fori_loop(0, NCH, _chunk, 0)

    plsc.subcore_barrier()
    pltpu.sync_copy(acc_s.at[pl.ds(row0, ROWS_PER_TILE)],
                    acc_out.at[c, pl.ds(row0, ROWS_PER_TILE)])


def _stage2b(wh, src, dstw, ex):
    mesh = plsc.VectorSubcoreMesh(core_axis_name="c", subcore_axis_name="s",
                                  num_cores=NC, num_subcores=NS)
    return pl.kernel(
        _sc_rows_body,
        out_type=jax.ShapeDtypeStruct((NC, NP, F), jnp.float32),
        mesh=mesh,
        compiler_params=pltpu.CompilerParams(needs_layout_passes=False),
        scratch_types=[
            pltpu.VMEM((CH,), jnp.int32),
            pltpu.VMEM((NBC, B), jnp.int32),
            pltpu.VMEM((CH,), jnp.float32),
            pltpu.VMEM((NBUF, B, F), jnp.float32),
            pltpu.SemaphoreType.DMA((NBUF,)),
            pltpu.SemaphoreType.DMA((NBUF,)),
            pltpu.VMEM_SHARED((NP, F), jnp.float32),
        ],
    )(wh, src, dstw, ex)


def _fin_body(acc_ref, den_ref, out_ref):
    i = pl.program_id(0)
    acc = acc_ref[0] + acc_ref[1]
    den = jnp.sum(den_ref[:, pl.ds(i * ROW_BLK, ROW_BLK)], axis=0)
    o = acc / (den[:, None] + 1e-16)
    out_ref[...] = jnp.where(o > 0.0, o, jnp.exp(jnp.minimum(o, 0.0)) - 1.0)


def _stage3(acc, den):
    return pl.pallas_call(
        _fin_body,
        grid=(GRID,),
        in_specs=[
            pl.BlockSpec((NC, ROW_BLK, F), lambda i: (0, i, 0)),
            pl.BlockSpec((NT, NP), lambda i: (0, 0)),
        ],
        out_specs=pl.BlockSpec((ROW_BLK, F), lambda i: (i, 0)),
        out_shape=jax.ShapeDtypeStruct((NP, F), jnp.float32),
    )(acc, den)


def kernel(x, edge_index, W, a):
    src = edge_index[0]
    dst = edge_index[1]
    wh, s1, s2 = _stage1(x, W, a)
    ex, den_parts = _stage2a(s1.reshape(N), s2.reshape(N), src, dst)
    dstw = dst.reshape(NT, NCH, NBC, B)
    acc = _stage2b(wh, src, dstw, ex)
    return _stage3(acc, den_parts.reshape(NT, NP))[:N]


# speedup vs baseline: 46.8077x; 1.0890x over previous
import jax
import jax.numpy as jnp
from jax import lax
from jax.experimental import pallas as pl
from jax.experimental.pallas import tpu as pltpu
from jax.experimental.pallas import tpu_sc as plsc

N = 10000
E = 320000
F = 128
ALPHA = 0.2

NC = 2
NS = 16
NT = NC * NS
EC = E // NT
B = 80
NBUF = 3
NP = 10240
ROWS_PER_TILE = NP // NS

MM_BLK = 1000
MM_GRID = N // MM_BLK
ROW_BLK = 1024
GRID = NP // ROW_BLK


def _mm_body(x_ref, w_ref, a_ref, wh_ref, s1_ref, s2_ref):
    xb = x_ref[...]
    wh = jnp.dot(xb, w_ref[...], preferred_element_type=jnp.float32)
    wh_ref[...] = wh
    s1_ref[...] = jnp.dot(wh, a_ref[:F], preferred_element_type=jnp.float32)
    s2_ref[...] = jnp.dot(wh, a_ref[F:], preferred_element_type=jnp.float32)


def _stage1(x, W, a):
    return pl.pallas_call(
        _mm_body,
        grid=(MM_GRID,),
        in_specs=[
            pl.BlockSpec((MM_BLK, F), lambda i: (i, 0)),
            pl.BlockSpec((F, F), lambda i: (0, 0)),
            pl.BlockSpec((2 * F, 1), lambda i: (0, 0)),
        ],
        out_specs=[
            pl.BlockSpec((MM_BLK, F), lambda i: (i, 0)),
            pl.BlockSpec((MM_BLK, 1), lambda i: (i, 0)),
            pl.BlockSpec((MM_BLK, 1), lambda i: (i, 0)),
        ],
        out_shape=[
            jax.ShapeDtypeStruct((N, F), jnp.float32),
            jax.ShapeDtypeStruct((N, 1), jnp.float32),
            jax.ShapeDtypeStruct((N, 1), jnp.float32),
        ],
    )(x, W, a)


def _sc_score_body(s1_hbm, s2_hbm, src_hbm, dst_hbm,
                   ex_out, den_out,
                   s1_v, s2_v, src_v, dst_v, ex_v, den_v):
    c = lax.axis_index("c")
    s = lax.axis_index("s")
    g = c * NS + s
    base = g * EC

    pltpu.sync_copy(s1_hbm, s1_v)
    pltpu.sync_copy(s2_hbm, s2_v)
    pltpu.sync_copy(src_hbm.at[pl.ds(base, EC)], src_v)
    pltpu.sync_copy(dst_hbm.at[pl.ds(base, EC)], dst_v)

    zv = jnp.zeros((16,), jnp.float32)

    def _zden(i, _):
        den_v[pl.ds(i * 16, 16)] = zv
        return 0
    lax.fori_loop(0, NP // 16, _zden, 0)

    def _score(i, _):
        d16 = dst_v[pl.ds(i * 16, 16)]
        t16 = src_v[pl.ds(i * 16, 16)]
        e = plsc.load_gather(s1_v, [d16]) + plsc.load_gather(s2_v, [t16])
        e = jnp.where(e >= 0.0, e, ALPHA * e)
        ex = jnp.exp(e)
        ex_v[pl.ds(i * 16, 16)] = ex
        plsc.addupdate_scatter(den_v, [d16], ex)
        return 0
    lax.fori_loop(0, EC // 16, _score, 0)

    pltpu.sync_copy(ex_v, ex_out.at[pl.ds(base, EC)])
    pltpu.sync_copy(den_v, den_out.at[pl.ds(g * NP, NP)])


def _stage2a(s1, s2, src, dst):
    mesh = plsc.VectorSubcoreMesh(core_axis_name="c", subcore_axis_name="s",
                                  num_cores=NC, num_subcores=NS)
    return pl.kernel(
        _sc_score_body,
        out_type=[
            jax.ShapeDtypeStruct((E,), jnp.float32),
            jax.ShapeDtypeStruct((NT * NP,), jnp.float32),
        ],
        mesh=mesh,
        compiler_params=pltpu.CompilerParams(needs_layout_passes=False),
        scratch_types=[
            pltpu.VMEM((N,), jnp.float32),
            pltpu.VMEM((N,), jnp.float32),
            pltpu.VMEM((EC,), jnp.int32),
            pltpu.VMEM((EC,), jnp.int32),
            pltpu.VMEM((EC,), jnp.float32),
            pltpu.VMEM((NP,), jnp.float32),
        ],
    )(s1, s2, src, dst)


CH = 2000
NCH = EC // CH
NBC = CH // B


def _sc_rows_body(wh_hbm, src_hbm, dstw_hbm, ex_hbm,
                  acc_out,
                  src_c, dstw_c, ex_c, rows_v, sems, ssems,
                  acc_s):
    c = lax.axis_index("c")
    s = lax.axis_index("s")
    g = c * NS + s
    base = g * EC

    zv = jnp.zeros((16,), jnp.float32)

    def _zrow(r, _):
        for f in range(F // 16):
            rows_v[0, r, pl.ds(f * 16, 16)] = zv
        return 0
    lax.fori_loop(0, B, _zrow, 0)
    row0 = s * ROWS_PER_TILE
    for t in range(ROWS_PER_TILE // B):
        pltpu.sync_copy(rows_v.at[0], acc_s.at[pl.ds(row0 + t * B, B)])

    plsc.subcore_barrier()

    def _issue(b, k):
        idx = src_c.at[pl.ds(b * B, B)]
        pltpu.async_copy(wh_hbm.at[idx], rows_v.at[k], sems.at[k])

    def _scale_rows(k, b):
        @plsc.parallel_loop(0, B, unroll=4)
        def _edge(kk):
            eidx = b * B + kk
            exb = plsc.load_gather(ex_c, [jnp.full((16,), eidx, jnp.int32)])
            for f in range(F // 16):
                sl = pl.ds(f * 16, 16)
                rows_v[k, kk, sl] = rows_v[k, kk, sl] * exb

    def _scatter(k, b):
        return pltpu.make_async_copy(rows_v.at[k], acc_s.at[dstw_c.at[b]],
                                     ssems.at[k])

    def _chunk(ch, _):
        pltpu.sync_copy(src_hbm.at[pl.ds(base + ch * CH, CH)], src_c)
        pltpu.sync_copy(ex_hbm.at[pl.ds(base + ch * CH, CH)], ex_c)
        pltpu.sync_copy(dstw_hbm.at[g, ch], dstw_c)

        for k in range(NBUF):
            _issue(k, k)

        def _outer(bb, _):
            for k in range(NBUF):
                b = bb * NBUF + k

                @pl.when(b < NBC)
                def _():
                    pltpu.make_async_copy(
                        wh_hbm.at[src_c.at[pl.ds(b * B, B)]],
                        rows_v.at[k], sems.at[k]).wait()
                    _scale_rows(k, b)
                    _scatter(k, b).start(add=True)

                    kp = (k + NBUF - 1) % NBUF

                    @pl.when(b >= 1)
                    def _():
                        _scatter(kp, b - 1).wait()

                        @pl.when(b - 1 + NBUF < NBC)
                        def _():
                            _issue(b - 1 + NBUF, kp)
            return 0
        lax.fori_loop(0, (NBC + NBUF - 1) // NBUF, _outer, 0)
        _scatter((NBC - 1) % NBUF, NBC - 1).wait()
        return 0
    lax.fori_loop(0, NCH, _chunk, 0)

    plsc.subcore_barrier()
    pltpu.sync_copy(acc_s.at[pl.ds(row0, ROWS_PER_TILE)],
                    acc_out.at[c, pl.ds(row0, ROWS_PER_TILE)])


def _stage2b(wh, src, dstw, ex):
    mesh = plsc.VectorSubcoreMesh(core_axis_name="c", subcore_axis_name="s",
                                  num_cores=NC, num_subcores=NS)
    return pl.kernel(
        _sc_rows_body,
        out_type=jax.ShapeDtypeStruct((NC, NP, F), jnp.float32),
        mesh=mesh,
        compiler_params=pltpu.CompilerParams(needs_layout_passes=False),
        scratch_types=[
            pltpu.VMEM((CH,), jnp.int32),
            pltpu.VMEM((NBC, B), jnp.int32),
            pltpu.VMEM((CH,), jnp.float32),
            pltpu.VMEM((NBUF, B, F), jnp.float32),
            pltpu.SemaphoreType.DMA((NBUF,)),
            pltpu.SemaphoreType.DMA((NBUF,)),
            pltpu.VMEM_SHARED((NP, F), jnp.float32),
        ],
    )(wh, src, dstw, ex)


def _fin_body(acc_ref, den_ref, out_ref):
    i = pl.program_id(0)
    acc = acc_ref[0] + acc_ref[1]
    den = jnp.sum(den_ref[:, pl.ds(i * ROW_BLK, ROW_BLK)], axis=0)
    o = acc / (den[:, None] + 1e-16)
    out_ref[...] = jnp.where(o > 0.0, o, jnp.exp(jnp.minimum(o, 0.0)) - 1.0)


def _stage3(acc, den):
    return pl.pallas_call(
        _fin_body,
        grid=(GRID,),
        in_specs=[
            pl.BlockSpec((NC, ROW_BLK, F), lambda i: (0, i, 0)),
            pl.BlockSpec((NT, NP), lambda i: (0, 0)),
        ],
        out_specs=pl.BlockSpec((ROW_BLK, F), lambda i: (i, 0)),
        out_shape=jax.ShapeDtypeStruct((NP, F), jnp.float32),
    )(acc, den)


def kernel(x, edge_index, W, a):
    src = edge_index[0]
    dst = edge_index[1]
    wh, s1, s2 = _stage1(x, W, a)
    ex, den_parts = _stage2a(s1.reshape(N), s2.reshape(N), src, dst)
    dstw = dst.reshape(NT, NCH, NBC, B)
    acc = _stage2b(wh, src, dstw, ex)
    return _stage3(acc, den_parts.reshape(NT, NP))[:N]


# speedup vs baseline: 47.5151x; 1.0151x over previous
import jax
import jax.numpy as jnp
from jax import lax
from jax.experimental import pallas as pl
from jax.experimental.pallas import tpu as pltpu
from jax.experimental.pallas import tpu_sc as plsc

N = 10000
E = 320000
F = 128
ALPHA = 0.2

NC = 2
NS = 16
NT = NC * NS
EC = E // NT
B = 80
NBUF = 4
NP = 10240
ROWS_PER_TILE = NP // NS

MM_BLK = 1000
MM_GRID = N // MM_BLK
ROW_BLK = 1024
GRID = NP // ROW_BLK


def _mm_body(x_ref, w_ref, a_ref, wh_ref, s1_ref, s2_ref):
    xb = x_ref[...]
    wh = jnp.dot(xb, w_ref[...], preferred_element_type=jnp.float32)
    wh_ref[...] = wh
    s1_ref[...] = jnp.dot(wh, a_ref[:F], preferred_element_type=jnp.float32)
    s2_ref[...] = jnp.dot(wh, a_ref[F:], preferred_element_type=jnp.float32)


def _stage1(x, W, a):
    return pl.pallas_call(
        _mm_body,
        grid=(MM_GRID,),
        in_specs=[
            pl.BlockSpec((MM_BLK, F), lambda i: (i, 0)),
            pl.BlockSpec((F, F), lambda i: (0, 0)),
            pl.BlockSpec((2 * F, 1), lambda i: (0, 0)),
        ],
        out_specs=[
            pl.BlockSpec((MM_BLK, F), lambda i: (i, 0)),
            pl.BlockSpec((MM_BLK, 1), lambda i: (i, 0)),
            pl.BlockSpec((MM_BLK, 1), lambda i: (i, 0)),
        ],
        out_shape=[
            jax.ShapeDtypeStruct((N, F), jnp.float32),
            jax.ShapeDtypeStruct((N, 1), jnp.float32),
            jax.ShapeDtypeStruct((N, 1), jnp.float32),
        ],
    )(x, W, a)


def _sc_score_body(s1_hbm, s2_hbm, src_hbm, dst_hbm,
                   ex_out, den_out,
                   s1_v, s2_v, src_v, dst_v, ex_v, den_v):
    c = lax.axis_index("c")
    s = lax.axis_index("s")
    g = c * NS + s
    base = g * EC

    pltpu.sync_copy(s1_hbm, s1_v)
    pltpu.sync_copy(s2_hbm, s2_v)
    pltpu.sync_copy(src_hbm.at[pl.ds(base, EC)], src_v)
    pltpu.sync_copy(dst_hbm.at[pl.ds(base, EC)], dst_v)

    zv = jnp.zeros((16,), jnp.float32)

    def _zden(i, _):
        den_v[pl.ds(i * 16, 16)] = zv
        return 0
    lax.fori_loop(0, NP // 16, _zden, 0)

    def _score(i, _):
        d16 = dst_v[pl.ds(i * 16, 16)]
        t16 = src_v[pl.ds(i * 16, 16)]
        e = plsc.load_gather(s1_v, [d16]) + plsc.load_gather(s2_v, [t16])
        e = jnp.where(e >= 0.0, e, ALPHA * e)
        ex = jnp.exp(e)
        ex_v[pl.ds(i * 16, 16)] = ex
        plsc.addupdate_scatter(den_v, [d16], ex)
        return 0
    lax.fori_loop(0, EC // 16, _score, 0)

    pltpu.sync_copy(ex_v, ex_out.at[pl.ds(base, EC)])
    pltpu.sync_copy(den_v, den_out.at[pl.ds(g * NP, NP)])


def _stage2a(s1, s2, src, dst):
    mesh = plsc.VectorSubcoreMesh(core_axis_name="c", subcore_axis_name="s",
                                  num_cores=NC, num_subcores=NS)
    return pl.kernel(
        _sc_score_body,
        out_type=[
            jax.ShapeDtypeStruct((E,), jnp.float32),
            jax.ShapeDtypeStruct((NT * NP,), jnp.float32),
        ],
        mesh=mesh,
        compiler_params=pltpu.CompilerParams(needs_layout_passes=False),
        scratch_types=[
            pltpu.VMEM((N,), jnp.float32),
            pltpu.VMEM((N,), jnp.float32),
            pltpu.VMEM((EC,), jnp.int32),
            pltpu.VMEM((EC,), jnp.int32),
            pltpu.VMEM((EC,), jnp.float32),
            pltpu.VMEM((NP,), jnp.float32),
        ],
    )(s1, s2, src, dst)


CH = 2000
NCH = EC // CH
NBC = CH // B


def _sc_rows_body(wh_hbm, src_hbm, dstw_hbm, ex_hbm,
                  acc_out,
                  src_c, dstw_c, ex_c, rows_v, sems, ssems,
                  acc_s):
    c = lax.axis_index("c")
    s = lax.axis_index("s")
    g = c * NS + s
    base = g * EC

    zv = jnp.zeros((16,), jnp.float32)

    def _zrow(r, _):
        for f in range(F // 16):
            rows_v[0, r, pl.ds(f * 16, 16)] = zv
        return 0
    lax.fori_loop(0, B, _zrow, 0)
    row0 = s * ROWS_PER_TILE
    for t in range(ROWS_PER_TILE // B):
        pltpu.sync_copy(rows_v.at[0], acc_s.at[pl.ds(row0 + t * B, B)])

    plsc.subcore_barrier()

    def _issue(b, k):
        idx = src_c.at[pl.ds(b * B, B)]
        pltpu.async_copy(wh_hbm.at[idx], rows_v.at[k], sems.at[k])

    def _scale_rows(k, b):
        @plsc.parallel_loop(0, B, unroll=8)
        def _edge(kk):
            eidx = b * B + kk
            exb = plsc.load_gather(ex_c, [jnp.full((16,), eidx, jnp.int32)])
            for f in range(F // 16):
                sl = pl.ds(f * 16, 16)
                rows_v[k, kk, sl] = rows_v[k, kk, sl] * exb

    def _scatter(k, b):
        return pltpu.make_async_copy(rows_v.at[k], acc_s.at[dstw_c.at[b]],
                                     ssems.at[k])

    def _chunk(ch, _):
        pltpu.sync_copy(src_hbm.at[pl.ds(base + ch * CH, CH)], src_c)
        pltpu.sync_copy(ex_hbm.at[pl.ds(base + ch * CH, CH)], ex_c)
        pltpu.sync_copy(dstw_hbm.at[g, ch], dstw_c)

        for k in range(NBUF):
            _issue(k, k)

        def _outer(bb, _):
            for k in range(NBUF):
                b = bb * NBUF + k

                @pl.when(b < NBC)
                def _():
                    pltpu.make_async_copy(
                        wh_hbm.at[src_c.at[pl.ds(b * B, B)]],
                        rows_v.at[k], sems.at[k]).wait()
                    _scale_rows(k, b)
                    _scatter(k, b).start(add=True)

                    kp = (k + NBUF - 1) % NBUF

                    @pl.when(b >= 1)
                    def _():
                        _scatter(kp, b - 1).wait()

                        @pl.when(b - 1 + NBUF < NBC)
                        def _():
                            _issue(b - 1 + NBUF, kp)
            return 0
        lax.fori_loop(0, (NBC + NBUF - 1) // NBUF, _outer, 0)
        _scatter((NBC - 1) % NBUF, NBC - 1).wait()
        return 0
    lax.fori_loop(0, NCH, _chunk, 0)

    plsc.subcore_barrier()
    pltpu.sync_copy(acc_s.at[pl.ds(row0, ROWS_PER_TILE)],
                    acc_out.at[c, pl.ds(row0, ROWS_PER_TILE)])


def _stage2b(wh, src, dstw, ex):
    mesh = plsc.VectorSubcoreMesh(core_axis_name="c", subcore_axis_name="s",
                                  num_cores=NC, num_subcores=NS)
    return pl.kernel(
        _sc_rows_body,
        out_type=jax.ShapeDtypeStruct((NC, NP, F), jnp.float32),
        mesh=mesh,
        compiler_params=pltpu.CompilerParams(needs_layout_passes=False),
        scratch_types=[
            pltpu.VMEM((CH,), jnp.int32),
            pltpu.VMEM((NBC, B), jnp.int32),
            pltpu.VMEM((CH,), jnp.float32),
            pltpu.VMEM((NBUF, B, F), jnp.float32),
            pltpu.SemaphoreType.DMA((NBUF,)),
            pltpu.SemaphoreType.DMA((NBUF,)),
            pltpu.VMEM_SHARED((NP, F), jnp.float32),
        ],
    )(wh, src, dstw, ex)


def _fin_body(acc_ref, den_ref, out_ref):
    i = pl.program_id(0)
    acc = acc_ref[0] + acc_ref[1]
    den = jnp.sum(den_ref[:, pl.ds(i * ROW_BLK, ROW_BLK)], axis=0)
    o = acc / (den[:, None] + 1e-16)
    out_ref[...] = jnp.where(o > 0.0, o, jnp.exp(jnp.minimum(o, 0.0)) - 1.0)


def _stage3(acc, den):
    return pl.pallas_call(
        _fin_body,
        grid=(GRID,),
        in_specs=[
            pl.BlockSpec((NC, ROW_BLK, F), lambda i: (0, i, 0)),
            pl.BlockSpec((NT, NP), lambda i: (0, 0)),
        ],
        out_specs=pl.BlockSpec((ROW_BLK, F), lambda i: (i, 0)),
        out_shape=jax.ShapeDtypeStruct((NP, F), jnp.float32),
    )(acc, den)


def kernel(x, edge_index, W, a):
    src = edge_index[0]
    dst = edge_index[1]
    wh, s1, s2 = _stage1(x, W, a)
    ex, den_parts = _stage2a(s1.reshape(N), s2.reshape(N), src, dst)
    dstw = dst.reshape(NT, NCH, NBC, B)
    acc = _stage2b(wh, src, dstw, ex)
    return _stage3(acc, den_parts.reshape(NT, NP))[:N]


# speedup vs baseline: 51.9591x; 1.0935x over previous
import jax
import jax.numpy as jnp
from jax import lax
from jax.experimental import pallas as pl
from jax.experimental.pallas import tpu as pltpu
from jax.experimental.pallas import tpu_sc as plsc

N = 10000
E = 320000
F = 128
ALPHA = 0.2

NC = 2
NS = 16
NT = NC * NS
EC = E // NT
B = 80
NBUF = 3
NBUFS = 2
NP = 10240
ROWS_PER_TILE = NP // NS

MM_BLK = 1000
MM_GRID = N // MM_BLK
ROW_BLK = 1024
GRID = NP // ROW_BLK


def _mm_body(x_ref, w_ref, a_ref, wh_ref, s1_ref, s2_ref):
    xb = x_ref[...]
    wh = jnp.dot(xb, w_ref[...], preferred_element_type=jnp.float32)
    wh_ref[...] = wh.astype(jnp.bfloat16)
    s1_ref[...] = jnp.dot(wh, a_ref[:F], preferred_element_type=jnp.float32)
    s2_ref[...] = jnp.dot(wh, a_ref[F:], preferred_element_type=jnp.float32)


def _stage1(x, W, a):
    return pl.pallas_call(
        _mm_body,
        grid=(MM_GRID,),
        in_specs=[
            pl.BlockSpec((MM_BLK, F), lambda i: (i, 0)),
            pl.BlockSpec((F, F), lambda i: (0, 0)),
            pl.BlockSpec((2 * F, 1), lambda i: (0, 0)),
        ],
        out_specs=[
            pl.BlockSpec((MM_BLK, F), lambda i: (i, 0)),
            pl.BlockSpec((MM_BLK, 1), lambda i: (i, 0)),
            pl.BlockSpec((MM_BLK, 1), lambda i: (i, 0)),
        ],
        out_shape=[
            jax.ShapeDtypeStruct((N, F), jnp.bfloat16),
            jax.ShapeDtypeStruct((N, 1), jnp.float32),
            jax.ShapeDtypeStruct((N, 1), jnp.float32),
        ],
    )(x, W, a)


def _sc_score_body(s1_hbm, s2_hbm, src_hbm, dst_hbm,
                   ex_out, den_out,
                   s1_v, s2_v, src_v, dst_v, ex_v, den_v):
    c = lax.axis_index("c")
    s = lax.axis_index("s")
    g = c * NS + s
    base = g * EC

    pltpu.sync_copy(s1_hbm, s1_v)
    pltpu.sync_copy(s2_hbm, s2_v)
    pltpu.sync_copy(src_hbm.at[pl.ds(base, EC)], src_v)
    pltpu.sync_copy(dst_hbm.at[pl.ds(base, EC)], dst_v)

    zv = jnp.zeros((16,), jnp.float32)

    def _zden(i, _):
        den_v[pl.ds(i * 16, 16)] = zv
        return 0
    lax.fori_loop(0, NP // 16, _zden, 0)

    def _score(i, _):
        d16 = dst_v[pl.ds(i * 16, 16)]
        t16 = src_v[pl.ds(i * 16, 16)]
        e = plsc.load_gather(s1_v, [d16]) + plsc.load_gather(s2_v, [t16])
        e = jnp.where(e >= 0.0, e, ALPHA * e)
        ex = jnp.exp(e)
        ex_v[pl.ds(i * 16, 16)] = ex
        plsc.addupdate_scatter(den_v, [d16], ex)
        return 0
    lax.fori_loop(0, EC // 16, _score, 0)

    pltpu.sync_copy(ex_v, ex_out.at[pl.ds(base, EC)])
    pltpu.sync_copy(den_v, den_out.at[pl.ds(g * NP, NP)])


def _stage2a(s1, s2, src, dst):
    mesh = plsc.VectorSubcoreMesh(core_axis_name="c", subcore_axis_name="s",
                                  num_cores=NC, num_subcores=NS)
    return pl.kernel(
        _sc_score_body,
        out_type=[
            jax.ShapeDtypeStruct((E,), jnp.float32),
            jax.ShapeDtypeStruct((NT * NP,), jnp.float32),
        ],
        mesh=mesh,
        compiler_params=pltpu.CompilerParams(needs_layout_passes=False),
        scratch_types=[
            pltpu.VMEM((N,), jnp.float32),
            pltpu.VMEM((N,), jnp.float32),
            pltpu.VMEM((EC,), jnp.int32),
            pltpu.VMEM((EC,), jnp.int32),
            pltpu.VMEM((EC,), jnp.float32),
            pltpu.VMEM((NP,), jnp.float32),
        ],
    )(s1, s2, src, dst)


CH = 2000
NCH = EC // CH
NBC = CH // B


def _sc_rows_body(wh_hbm, src_hbm, dstw_hbm, ex_hbm,
                  acc_out,
                  src_c, dstw_c, ex_c, rows_bf, rows_v, sems, ssems,
                  acc_s):
    c = lax.axis_index("c")
    s = lax.axis_index("s")
    g = c * NS + s
    base = g * EC

    zv = jnp.zeros((16,), jnp.float32)

    def _zrow(r, _):
        for f in range(F // 16):
            rows_v[0, r, pl.ds(f * 16, 16)] = zv
        return 0
    lax.fori_loop(0, B, _zrow, 0)
    row0 = s * ROWS_PER_TILE
    for t in range(ROWS_PER_TILE // B):
        pltpu.sync_copy(rows_v.at[0], acc_s.at[pl.ds(row0 + t * B, B)])

    plsc.subcore_barrier()

    def _issue(b, k):
        idx = src_c.at[pl.ds(b * B, B)]
        pltpu.async_copy(wh_hbm.at[idx], rows_bf.at[k], sems.at[k])

    def _scale_rows(kb, ks, b):
        @plsc.parallel_loop(0, B, unroll=8)
        def _edge(kk):
            eidx = b * B + kk
            exb = plsc.load_gather(ex_c, [jnp.full((16,), eidx, jnp.int32)])
            for f in range(F // 32):
                x = plsc.bitcast(rows_bf[kb, kk, pl.ds(f * 16, 16)],
                                 jnp.bfloat16)
                lo, hi = plsc.unpack(x, format=plsc.PackFormat.INTERLEAVED)
                rows_v[ks, kk, pl.ds(f * 32, 16)] = lo * exb
                rows_v[ks, kk, pl.ds(f * 32 + 16, 16)] = hi * exb

    def _scatter(k, b):
        return pltpu.make_async_copy(rows_v.at[k], acc_s.at[dstw_c.at[b]],
                                     ssems.at[k])

    def _chunk(ch, _):
        pltpu.sync_copy(src_hbm.at[pl.ds(base + ch * CH, CH)], src_c)
        pltpu.sync_copy(ex_hbm.at[pl.ds(base + ch * CH, CH)], ex_c)
        pltpu.sync_copy(dstw_hbm.at[g, ch], dstw_c)

        for k in range(NBUF):
            _issue(k, k)

        LCM = NBUF * NBUFS if NBUF % NBUFS else NBUF

        def _outer(bb, _):
            for k in range(LCM):
                b = bb * LCM + k
                kb = k % NBUF
                ks = k % NBUFS

                @pl.when(b < NBC)
                def _():
                    pltpu.make_async_copy(
                        wh_hbm.at[src_c.at[pl.ds(b * B, B)]],
                        rows_bf.at[kb], sems.at[kb]).wait()

                    @pl.when(b >= NBUFS)
                    def _():
                        _scatter(ks, b - NBUFS).wait()
                    _scale_rows(kb, ks, b)
                    _scatter(ks, b).start(add=True)

                    @pl.when(b + NBUF < NBC)
                    def _():
                        _issue(b + NBUF, kb)
            return 0
        lax.fori_loop(0, (NBC + LCM - 1) // LCM, _outer, 0)
        for d in range(NBUFS):
            _scatter((NBC - NBUFS + d) % NBUFS, NBC - NBUFS + d).wait()
        return 0
    lax.fori_loop(0, NCH, _chunk, 0)

    plsc.subcore_barrier()
    pltpu.sync_copy(acc_s.at[pl.ds(row0, ROWS_PER_TILE)],
                    acc_out.at[c, pl.ds(row0, ROWS_PER_TILE)])


def _stage2b(wh, src, dstw, ex):
    mesh = plsc.VectorSubcoreMesh(core_axis_name="c", subcore_axis_name="s",
                                  num_cores=NC, num_subcores=NS)
    return pl.kernel(
        _sc_rows_body,
        out_type=jax.ShapeDtypeStruct((NC, NP, F), jnp.float32),
        mesh=mesh,
        compiler_params=pltpu.CompilerParams(needs_layout_passes=False,
                                             use_tc_tiling_on_sc=False),
        scratch_types=[
            pltpu.VMEM((CH,), jnp.int32),
            pltpu.VMEM((NBC, B), jnp.int32),
            pltpu.VMEM((CH,), jnp.float32),
            pltpu.VMEM((NBUF, B, F // 2), jnp.int32),
            pltpu.VMEM((NBUFS, B, F), jnp.float32),
            pltpu.SemaphoreType.DMA((NBUF,)),
            pltpu.SemaphoreType.DMA((NBUFS,)),
            pltpu.VMEM_SHARED((NP, F), jnp.float32),
        ],
    )(wh, src, dstw, ex)


def _fin_body(acc_ref, den_ref, out_ref):
    i = pl.program_id(0)
    acc = acc_ref[0] + acc_ref[1]
    den = jnp.sum(den_ref[:, pl.ds(i * ROW_BLK, ROW_BLK)], axis=0)
    o = acc / (den[:, None] + 1e-16)
    out_ref[...] = jnp.where(o > 0.0, o, jnp.exp(jnp.minimum(o, 0.0)) - 1.0)


def _stage3(acc, den):
    return pl.pallas_call(
        _fin_body,
        grid=(GRID,),
        in_specs=[
            pl.BlockSpec((NC, ROW_BLK, F), lambda i: (0, i, 0)),
            pl.BlockSpec((NT, NP), lambda i: (0, 0)),
        ],
        out_specs=pl.BlockSpec((ROW_BLK, F), lambda i: (i, 0)),
        out_shape=jax.ShapeDtypeStruct((NP, F), jnp.float32),
    )(acc, den)


def kernel(x, edge_index, W, a):
    src = edge_index[0]
    dst = edge_index[1]
    wh_bf, s1, s2 = _stage1(x, W, a)
    wh_bf = (wh_bf.reshape(N, F // 32, 2, 16)
             .transpose(0, 1, 3, 2).reshape(N, F))
    wh_bf = lax.bitcast_convert_type(wh_bf.reshape(N, F // 2, 2),
                                     jnp.int32)
    ex, den_parts = _stage2a(s1.reshape(N), s2.reshape(N), src, dst)
    dstw = dst.reshape(NT, NCH, NBC, B)
    acc = _stage2b(wh_bf, src, dstw, ex)
    return _stage3(acc, den_parts.reshape(NT, NP))[:N]
